# Initial kernel scaffold; baseline (speedup 1.0000x reference)
import functools

import jax
import jax.numpy as jnp
from jax import lax
from jax.experimental import pallas as pl
from jax.experimental.pallas import tpu as pltpu
from jax.experimental.pallas import tpu_sc as plsc

N = 10000
E = 640000
T = 8
F_IN = 14
D = 64
H = 128

NC = 2
NS = 16
CHUNK = 80
RSLAB = 400
NSLAB = N // RSLAB
T_PER_CORE = T // NC

_mesh = plsc.VectorSubcoreMesh(core_axis_name="c", subcore_axis_name="s")


@functools.partial(
    pl.kernel,
    out_type=jax.ShapeDtypeStruct((NC, N), jnp.float32),
    mesh=_mesh,
    scratch_types=[
        pltpu.VMEM((CHUNK,), jnp.int32),
        pltpu.VMEM((CHUNK,), jnp.float32),
        pltpu.VMEM((RSLAB,), jnp.float32),
        pltpu.VMEM_SHARED((N,), jnp.float32),
    ],
)
def _deg_kernel(col_hbm, ew_hbm, out_hbm, colv, ewv, slab, degsh):
    c = lax.axis_index("c")
    s = lax.axis_index("s")
    for j in range(RSLAB // 16):
        slab[pl.ds(j * 16, 16)] = jnp.zeros((16,), jnp.float32)
    for kk in range(2):
        k = s + kk * NS
        @pl.when(k < NSLAB)
        def _():
            pltpu.sync_copy(slab, degsh.at[pl.ds(k * RSLAB, RSLAB)])
    plsc.subcore_barrier()

    ec = E // NC
    et = ec // NS
    nch = et // CHUNK

    def chunk(i, carry):
        base = c * ec + s * et + i * CHUNK
        pltpu.sync_copy(col_hbm.at[pl.ds(base, CHUNK)], colv)
        pltpu.sync_copy(ew_hbm.at[pl.ds(base, CHUNK)], ewv)
        pltpu.sync_copy(ewv, degsh.at[colv], add=True)
        return carry

    lax.fori_loop(0, nch, chunk, 0)
    plsc.subcore_barrier()
    for kk in range(2):
        k = s + kk * NS
        @pl.when(k < NSLAB)
        def _():
            pltpu.sync_copy(degsh.at[pl.ds(k * RSLAB, RSLAB)], slab)
            pltpu.sync_copy(slab, out_hbm.at[c].at[pl.ds(k * RSLAB, RSLAB)])


@functools.partial(
    pl.kernel,
    out_type=jax.ShapeDtypeStruct((T * N, D), jnp.float32),
    mesh=_mesh,
    scratch_types=[
        pltpu.VMEM((CHUNK,), jnp.int32),
        pltpu.VMEM((CHUNK,), jnp.int32),
        pltpu.VMEM((CHUNK,), jnp.float32),
        pltpu.VMEM((CHUNK, D), jnp.float32),
        pltpu.VMEM((RSLAB, D), jnp.float32),
        pltpu.VMEM_SHARED((N, D), jnp.float32),
        pltpu.SemaphoreType.DMA,
    ],
)
def _agg_kernel(tab_hbm, row_hbm, col_hbm, ew_hbm, out_hbm,
                rowv, colv, ewv, gbuf, slab, aggsh, sem):
    c = lax.axis_index("c")
    s = lax.axis_index("s")
    et = E // NS
    nch = et // CHUNK

    for tl in range(T_PER_CORE):
        toff = c * (T_PER_CORE * N) + tl * N
        for kk in range(2):
            k = s + kk * NS
            @pl.when(k < NSLAB)
            def _():
                pltpu.sync_copy(tab_hbm.at[pl.ds(toff + k * RSLAB, RSLAB)], slab)
                pltpu.sync_copy(slab, aggsh.at[pl.ds(k * RSLAB, RSLAB)])
        plsc.subcore_barrier()

        def chunk(i, carry):
            base = s * et + i * CHUNK
            pltpu.sync_copy(row_hbm.at[pl.ds(base, CHUNK)], rowv)
            pltpu.sync_copy(col_hbm.at[pl.ds(base, CHUNK)], colv)
            pltpu.sync_copy(ew_hbm.at[pl.ds(base, CHUNK)], ewv)
            for j in range(CHUNK // 16):
                rowv[pl.ds(j * 16, 16)] = rowv[pl.ds(j * 16, 16)] + toff
            pltpu.async_copy(tab_hbm.at[rowv], gbuf, sem).wait()

            def scale(e, cc):
                w = ewv[e]
                for j in range(D // 16):
                    gbuf[e, pl.ds(j * 16, 16)] = gbuf[e, pl.ds(j * 16, 16)] * w
                return cc

            lax.fori_loop(0, CHUNK, scale, 0)
            pltpu.sync_copy(gbuf, aggsh.at[colv], add=True)
            return carry

        lax.fori_loop(0, nch, chunk, 0)
        plsc.subcore_barrier()
        for kk in range(2):
            k = s + kk * NS
            @pl.when(k < NSLAB)
            def _():
                pltpu.sync_copy(aggsh.at[pl.ds(k * RSLAB, RSLAB)], slab)
                pltpu.sync_copy(slab, out_hbm.at[pl.ds(toff + k * RSLAB, RSLAB)])
        plsc.subcore_barrier()


BR = 2000
NB = T * N // BR
NBN = N // BR


def _prep_body(x_ref, w1_ref, dinv_ref, o_ref):
    h = jnp.dot(x_ref[...], w1_ref[...], preferred_element_type=jnp.float32)
    o_ref[...] = h * dinv_ref[...]


def _prep_call(xf, W1, dinv):
    return pl.pallas_call(
        _prep_body,
        grid=(NB,),
        in_specs=[
            pl.BlockSpec((BR, F_IN), lambda i: (i, 0)),
            pl.BlockSpec((F_IN, D), lambda i: (0, 0)),
            pl.BlockSpec((BR, 1), lambda i: (i % NBN, 0)),
        ],
        out_specs=pl.BlockSpec((BR, D), lambda i: (i, 0)),
        out_shape=jax.ShapeDtypeStruct((T * N, D), jnp.float32),
    )(xf, W1, dinv)


def _mid_body(a_ref, dinv_ref, b1_ref, w2_ref, o_ref):
    h = jnp.maximum(a_ref[...] * dinv_ref[...] + b1_ref[...], 0.0)
    o_ref[...] = jnp.dot(h, w2_ref[...], preferred_element_type=jnp.float32) * dinv_ref[...]


def _mid_call(a1, dinv, b1, W2):
    return pl.pallas_call(
        _mid_body,
        grid=(NB,),
        in_specs=[
            pl.BlockSpec((BR, D), lambda i: (i, 0)),
            pl.BlockSpec((BR, 1), lambda i: (i % NBN, 0)),
            pl.BlockSpec((1, D), lambda i: (0, 0)),
            pl.BlockSpec((D, D), lambda i: (0, 0)),
        ],
        out_specs=pl.BlockSpec((BR, D), lambda i: (i, 0)),
        out_shape=jax.ShapeDtypeStruct((T * N, D), jnp.float32),
    )(a1, dinv, b1, W2)


def _final_body(a_ref, dinv_ref, b2_ref, bt_ref, wi_ref, wh_ref, bl_ref,
                wm1_ref, bm1_ref, wm2_ref, bm2_ref, o_ref):
    dv = dinv_ref[...]
    add = b2_ref[...] + bt_ref[...]
    h = jnp.zeros((BR, H), jnp.float32)
    cst = jnp.zeros((BR, H), jnp.float32)
    wi = wi_ref[...]
    wh = wh_ref[...]
    bl = bl_ref[...]
    for t in range(T):
        et = a_ref[t] * dv + add
        g = (jnp.dot(et, wi, preferred_element_type=jnp.float32)
             + jnp.dot(h, wh, preferred_element_type=jnp.float32) + bl)
        gi = jax.nn.sigmoid(g[:, 0:H])
        gf = jax.nn.sigmoid(g[:, H:2 * H])
        gg = jnp.tanh(g[:, 2 * H:3 * H])
        go = jax.nn.sigmoid(g[:, 3 * H:4 * H])
        cst = gf * cst + gi * gg
        h = go * jnp.tanh(cst)
    z = jnp.maximum(jnp.dot(h, wm1_ref[...], preferred_element_type=jnp.float32)
                    + bm1_ref[...], 0.0)
    o_ref[...] = jnp.dot(z, wm2_ref[...], preferred_element_type=jnp.float32) + bm2_ref[...]


def _final_call(a2, dinv, b2, bias_table, Wi, Wh, b_lstm, Wm1, bm1, Wm2, bm2):
    return pl.pallas_call(
        _final_body,
        grid=(NBN,),
        in_specs=[
            pl.BlockSpec((T, BR, D), lambda i: (0, i, 0)),
            pl.BlockSpec((BR, 1), lambda i: (i, 0)),
            pl.BlockSpec((1, D), lambda i: (0, 0)),
            pl.BlockSpec((BR, D), lambda i: (i, 0)),
            pl.BlockSpec((D, 4 * H), lambda i: (0, 0)),
            pl.BlockSpec((H, 4 * H), lambda i: (0, 0)),
            pl.BlockSpec((1, 4 * H), lambda i: (0, 0)),
            pl.BlockSpec((H, D), lambda i: (0, 0)),
            pl.BlockSpec((1, D), lambda i: (0, 0)),
            pl.BlockSpec((D, 1), lambda i: (0, 0)),
            pl.BlockSpec((1, 1), lambda i: (0, 0)),
        ],
        out_specs=pl.BlockSpec((BR, 1), lambda i: (i, 0)),
        out_shape=jax.ShapeDtypeStruct((N, 1), jnp.float32),
    )(a2, dinv, b2, bias_table, Wi, Wh, b_lstm, Wm1, bm1, Wm2, bm2)


def kernel(x, edge_index, edge_weight, W1, b1, W2, b2, bias_table,
           Wi, Wh, b_lstm, Wm1, bm1, Wm2, bm2):
    row = edge_index[0]
    col = edge_index[1]
    deg_parts = _deg_kernel(col, edge_weight)
    deg = deg_parts[0] + deg_parts[1] + 1.0
    dinv = jnp.where(deg > 0, lax.rsqrt(jnp.maximum(deg, 1e-12)), 0.0)[:, None]

    xf = x.reshape(T * N, F_IN)
    hs1 = _prep_call(xf, W1, dinv)
    a1 = _agg_kernel(hs1, row, col, edge_weight)
    hs2 = _mid_call(a1, dinv, b1.reshape(1, D), W2)
    a2 = _agg_kernel(hs2, row, col, edge_weight)
    out = _final_call(a2.reshape(T, N, D), dinv, b2.reshape(1, D), bias_table,
                      Wi, Wh, b_lstm.reshape(1, 4 * H), Wm1, bm1.reshape(1, D // 2),
                      Wm2, bm2.reshape(1, 1))
    return out[:, 0]


# speedup vs baseline: 13.2489x; 13.2489x over previous
import functools

import jax
import jax.numpy as jnp
from jax import lax
from jax.experimental import pallas as pl
from jax.experimental.pallas import tpu as pltpu
from jax.experimental.pallas import tpu_sc as plsc

N = 10000
E = 640000
T = 8
F_IN = 14
D = 64
H = 128

NC = 2
NS = 16
CHUNK = 128
NCHUNKS = E // CHUNK
RSLAB = 400
NSLAB = N // RSLAB
NP1 = 10240
DSLAB = NP1 // NS
T_PER_CORE = T // NC

_mesh = plsc.VectorSubcoreMesh(core_axis_name="c", subcore_axis_name="s")


@functools.partial(
    pl.kernel,
    out_type=jax.ShapeDtypeStruct((NC * NP1,), jnp.float32),
    mesh=_mesh,
    scratch_types=[
        pltpu.VMEM((CHUNK,), jnp.int32),
        pltpu.VMEM((CHUNK,), jnp.float32),
        pltpu.VMEM((DSLAB,), jnp.float32),
        pltpu.VMEM_SHARED((NP1,), jnp.float32),
    ],
)
def _deg_kernel(col_hbm, ew_hbm, out_hbm, colv, ewv, slab, degsh):
    c = lax.axis_index("c")
    s = lax.axis_index("s")
    for j in range(DSLAB // 16):
        slab[pl.ds(j * 16, 16)] = jnp.zeros((16,), jnp.float32)
    pltpu.sync_copy(slab, degsh.at[pl.ds(s * DSLAB, DSLAB)])
    plsc.subcore_barrier()

    w = s * NC + c
    nit = NCHUNKS // (NC * NS) + 1

    def chunk(i, carry):
        j = w + i * (NC * NS)
        @pl.when(j < NCHUNKS)
        def _():
            base = j * CHUNK
            pltpu.sync_copy(col_hbm.at[pl.ds(base, CHUNK)], colv)
            pltpu.sync_copy(ew_hbm.at[pl.ds(base, CHUNK)], ewv)
            pltpu.sync_copy(ewv, degsh.at[colv], add=True)
        return carry

    lax.fori_loop(0, nit, chunk, 0)
    plsc.subcore_barrier()
    pltpu.sync_copy(degsh.at[pl.ds(s * DSLAB, DSLAB)], slab)
    pltpu.sync_copy(slab, out_hbm.at[pl.ds(c * NP1 + s * DSLAB, DSLAB)])


@functools.partial(
    pl.kernel,
    out_type=jax.ShapeDtypeStruct((T * N, D), jnp.float32),
    mesh=_mesh,
    scratch_types=[
        pltpu.VMEM((CHUNK,), jnp.int32),
        pltpu.VMEM((CHUNK,), jnp.int32),
        pltpu.VMEM((CHUNK,), jnp.float32),
        pltpu.VMEM((CHUNK, D), jnp.float32),
        pltpu.VMEM((RSLAB, D), jnp.float32),
        pltpu.VMEM_SHARED((N, D), jnp.float32),
        pltpu.SemaphoreType.DMA,
    ],
    compiler_params=pltpu.CompilerParams(use_tc_tiling_on_sc=False),
)
def _agg_kernel(tab_hbm, row_hbm, col_hbm, ew_hbm, out_hbm,
                rowv, colv, ewv, gbuf, slab, aggsh, sem):
    c = lax.axis_index("c")
    s = lax.axis_index("s")

    for tl in range(T_PER_CORE):
        toff = c * (T_PER_CORE * N) + tl * N
        for kk in range(2):
            k = s + kk * NS
            @pl.when(k < NSLAB)
            def _():
                pltpu.sync_copy(tab_hbm.at[pl.ds(toff + k * RSLAB, RSLAB)], slab)
                pltpu.sync_copy(slab, aggsh.at[pl.ds(k * RSLAB, RSLAB)])
        plsc.subcore_barrier()

        def chunk(i, carry):
            j = s + i * NS
            @pl.when(j < NCHUNKS)
            def _():
                base = j * CHUNK
                pltpu.sync_copy(row_hbm.at[pl.ds(base, CHUNK)], rowv)
                pltpu.sync_copy(col_hbm.at[pl.ds(base, CHUNK)], colv)
                pltpu.sync_copy(ew_hbm.at[pl.ds(base, CHUNK)], ewv)
                for jj in range(CHUNK // 16):
                    rowv[pl.ds(jj * 16, 16)] = rowv[pl.ds(jj * 16, 16)] + toff
                pltpu.async_copy(tab_hbm.at[rowv], gbuf, sem).wait()

                def scale(e16, cc):
                    wv = ewv[pl.ds(e16 * 16, 16)]
                    for l in range(16):
                        e = e16 * 16 + l
                        w = wv[l]
                        for jj in range(D // 16):
                            gbuf[e, pl.ds(jj * 16, 16)] = gbuf[e, pl.ds(jj * 16, 16)] * w
                    return cc

                lax.fori_loop(0, CHUNK // 16, scale, 0)
                pltpu.sync_copy(gbuf, aggsh.at[colv], add=True)
            return carry

        lax.fori_loop(0, NCHUNKS // NS + 1, chunk, 0)
        plsc.subcore_barrier()
        for kk in range(2):
            k = s + kk * NS
            @pl.when(k < NSLAB)
            def _():
                pltpu.sync_copy(aggsh.at[pl.ds(k * RSLAB, RSLAB)], slab)
                pltpu.sync_copy(slab, out_hbm.at[pl.ds(toff + k * RSLAB, RSLAB)])
        plsc.subcore_barrier()


BR = 2000
NB = T * N // BR
NBN = N // BR


def _prep_body(x_ref, w1_ref, dinv_ref, o_ref):
    h = jnp.dot(x_ref[...], w1_ref[...], preferred_element_type=jnp.float32)
    o_ref[...] = h * dinv_ref[...]


def _prep_call(xf, W1, dinv):
    return pl.pallas_call(
        _prep_body,
        grid=(NB,),
        in_specs=[
            pl.BlockSpec((BR, F_IN), lambda i: (i, 0)),
            pl.BlockSpec((F_IN, D), lambda i: (0, 0)),
            pl.BlockSpec((BR, 1), lambda i: (i % NBN, 0)),
        ],
        out_specs=pl.BlockSpec((BR, D), lambda i: (i, 0)),
        out_shape=jax.ShapeDtypeStruct((T * N, D), jnp.float32),
    )(xf, W1, dinv)


def _mid_body(a_ref, dinv_ref, b1_ref, w2_ref, o_ref):
    h = jnp.maximum(a_ref[...] * dinv_ref[...] + b1_ref[...], 0.0)
    o_ref[...] = jnp.dot(h, w2_ref[...], preferred_element_type=jnp.float32) * dinv_ref[...]


def _mid_call(a1, dinv, b1, W2):
    return pl.pallas_call(
        _mid_body,
        grid=(NB,),
        in_specs=[
            pl.BlockSpec((BR, D), lambda i: (i, 0)),
            pl.BlockSpec((BR, 1), lambda i: (i % NBN, 0)),
            pl.BlockSpec((1, D), lambda i: (0, 0)),
            pl.BlockSpec((D, D), lambda i: (0, 0)),
        ],
        out_specs=pl.BlockSpec((BR, D), lambda i: (i, 0)),
        out_shape=jax.ShapeDtypeStruct((T * N, D), jnp.float32),
    )(a1, dinv, b1, W2)


def _final_body(a_ref, dinv_ref, b2_ref, bt_ref, wi_ref, wh_ref, bl_ref,
                wm1_ref, bm1_ref, wm2_ref, bm2_ref, o_ref):
    dv = dinv_ref[...]
    add = b2_ref[...] + bt_ref[...]
    h = jnp.zeros((BR, H), jnp.float32)
    cst = jnp.zeros((BR, H), jnp.float32)
    wi = wi_ref[...]
    wh = wh_ref[...]
    bl = bl_ref[...]
    for t in range(T):
        et = a_ref[t] * dv + add
        g = (jnp.dot(et, wi, preferred_element_type=jnp.float32)
             + jnp.dot(h, wh, preferred_element_type=jnp.float32) + bl)
        gi = jax.nn.sigmoid(g[:, 0:H])
        gf = jax.nn.sigmoid(g[:, H:2 * H])
        gg = jnp.tanh(g[:, 2 * H:3 * H])
        go = jax.nn.sigmoid(g[:, 3 * H:4 * H])
        cst = gf * cst + gi * gg
        h = go * jnp.tanh(cst)
    z = jnp.maximum(jnp.dot(h, wm1_ref[...], preferred_element_type=jnp.float32)
                    + bm1_ref[...], 0.0)
    o_ref[...] = jnp.dot(z, wm2_ref[...], preferred_element_type=jnp.float32) + bm2_ref[...]


def _final_call(a2, dinv, b2, bias_table, Wi, Wh, b_lstm, Wm1, bm1, Wm2, bm2):
    return pl.pallas_call(
        _final_body,
        grid=(NBN,),
        in_specs=[
            pl.BlockSpec((T, BR, D), lambda i: (0, i, 0)),
            pl.BlockSpec((BR, 1), lambda i: (i, 0)),
            pl.BlockSpec((1, D), lambda i: (0, 0)),
            pl.BlockSpec((BR, D), lambda i: (i, 0)),
            pl.BlockSpec((D, 4 * H), lambda i: (0, 0)),
            pl.BlockSpec((H, 4 * H), lambda i: (0, 0)),
            pl.BlockSpec((1, 4 * H), lambda i: (0, 0)),
            pl.BlockSpec((H, D), lambda i: (0, 0)),
            pl.BlockSpec((1, D), lambda i: (0, 0)),
            pl.BlockSpec((D, 1), lambda i: (0, 0)),
            pl.BlockSpec((1, 1), lambda i: (0, 0)),
        ],
        out_specs=pl.BlockSpec((BR, 1), lambda i: (i, 0)),
        out_shape=jax.ShapeDtypeStruct((N, 1), jnp.float32),
    )(a2, dinv, b2, bias_table, Wi, Wh, b_lstm, Wm1, bm1, Wm2, bm2)


def kernel(x, edge_index, edge_weight, W1, b1, W2, b2, bias_table,
           Wi, Wh, b_lstm, Wm1, bm1, Wm2, bm2):
    row = edge_index[0]
    col = edge_index[1]
    deg_parts = _deg_kernel(col, edge_weight).reshape(NC, NP1)
    deg = (deg_parts[0] + deg_parts[1] + 1.0)[:N]
    dinv = jnp.where(deg > 0, lax.rsqrt(jnp.maximum(deg, 1e-12)), 0.0)[:, None]

    xf = x.reshape(T * N, F_IN)
    hs1 = _prep_call(xf, W1, dinv)
    a1 = _agg_kernel(hs1, row, col, edge_weight)
    hs2 = _mid_call(a1, dinv, b1.reshape(1, D), W2)
    a2 = _agg_kernel(hs2, row, col, edge_weight)
    out = _final_call(a2.reshape(T, N, D), dinv, b2.reshape(1, D), bias_table,
                      Wi, Wh, b_lstm.reshape(1, 4 * H), Wm1, bm1.reshape(1, H // 2),
                      Wm2, bm2.reshape(1, 1))
    return out[:, 0]


# speedup vs baseline: 27.3377x; 2.0634x over previous
import functools

import jax
import jax.numpy as jnp
from jax import lax
from jax.experimental import pallas as pl
from jax.experimental.pallas import tpu as pltpu
from jax.experimental.pallas import tpu_sc as plsc

N = 10000
E = 640000
T = 8
F_IN = 14
D = 64
H = 128

NC = 2
NS = 16
CHUNK = 128
EP = 655360
NCHUNKS = EP // CHUNK
CPT = NCHUNKS // NS
IB = 8
IBE = IB * CHUNK
NGRP = CPT // IB
NGP2 = NGRP // 2
NBUF = 4
RSLAB = 400
NSLAB = N // RSLAB
NP1 = 10240
DSLAB = NP1 // NS
T_PER_CORE = T // NC

_mesh = plsc.VectorSubcoreMesh(core_axis_name="c", subcore_axis_name="s")


@functools.partial(
    pl.kernel,
    out_type=jax.ShapeDtypeStruct((NC * NP1,), jnp.float32),
    mesh=_mesh,
    scratch_types=[
        pltpu.VMEM((CHUNK,), jnp.int32),
        pltpu.VMEM((CHUNK,), jnp.float32),
        pltpu.VMEM((DSLAB,), jnp.float32),
        pltpu.VMEM_SHARED((NP1,), jnp.float32),
    ],
)
def _deg_kernel(col_hbm, ew_hbm, out_hbm, colv, ewv, slab, degsh):
    c = lax.axis_index("c")
    s = lax.axis_index("s")
    for j in range(DSLAB // 16):
        slab[pl.ds(j * 16, 16)] = jnp.zeros((16,), jnp.float32)
    pltpu.sync_copy(slab, degsh.at[pl.ds(s * DSLAB, DSLAB)])
    plsc.subcore_barrier()

    w = s * NC + c
    nit = NCHUNKS // (NC * NS)

    def chunk(i, carry):
        base = (w * nit + i) * CHUNK
        pltpu.sync_copy(col_hbm.at[pl.ds(base, CHUNK)], colv)
        pltpu.sync_copy(ew_hbm.at[pl.ds(base, CHUNK)], ewv)
        pltpu.sync_copy(ewv, degsh.at[colv], add=True)
        return carry

    lax.fori_loop(0, nit, chunk, 0)
    plsc.subcore_barrier()
    pltpu.sync_copy(degsh.at[pl.ds(s * DSLAB, DSLAB)], slab)
    pltpu.sync_copy(slab, out_hbm.at[pl.ds(c * NP1 + s * DSLAB, DSLAB)])


@functools.partial(
    pl.kernel,
    out_type=jax.ShapeDtypeStruct((T * N, D), jnp.float32),
    mesh=_mesh,
    scratch_types=[
        pltpu.VMEM((2, IBE), jnp.int32),
        pltpu.VMEM((2, IB, CHUNK), jnp.int32),
        pltpu.VMEM((2, IBE), jnp.float32),
        pltpu.VMEM((NBUF, CHUNK, D), jnp.float32),
        pltpu.VMEM((RSLAB, D), jnp.float32),
        pltpu.VMEM_SHARED((N, D), jnp.float32),
        pltpu.SemaphoreType.DMA,
        pltpu.SemaphoreType.DMA,
        pltpu.SemaphoreType.DMA,
        pltpu.SemaphoreType.DMA,
        pltpu.SemaphoreType.DMA,
        pltpu.SemaphoreType.DMA,
        pltpu.SemaphoreType.DMA,
        pltpu.SemaphoreType.DMA,
        pltpu.SemaphoreType.DMA,
        pltpu.SemaphoreType.DMA,
    ],
    compiler_params=pltpu.CompilerParams(use_tc_tiling_on_sc=False),
)
def _agg_kernel(tab_hbm, row_hbm, col_hbm, ew_hbm, out_hbm,
                rowb, colb, ewb, gbuf, slab, aggsh,
                is0, is1, gs0, gs1, gs2, gs3, ss0, ss1, ss2, ss3):
    c = lax.axis_index("c")
    s = lax.axis_index("s")
    isem = (is0, is1)
    gsem = (gs0, gs1, gs2, gs3)
    ssem = (ss0, ss1, ss2, ss3)
    ebase = s * (CPT * CHUNK)
    cbase = s * CPT

    def issue_idx(g, q):
        eb = ebase + g * IBE
        pltpu.async_copy(row_hbm.at[pl.ds(eb, IBE)], rowb.at[q], isem[q])
        pltpu.async_copy(col_hbm.at[pl.ds(cbase + g * IB, IB)], colb.at[q], isem[q])
        pltpu.async_copy(ew_hbm.at[pl.ds(eb, IBE)], ewb.at[q], isem[q])

    def wait_idx(g, q):
        eb = ebase + g * IBE
        pltpu.make_async_copy(row_hbm.at[pl.ds(eb, IBE)], rowb.at[q], isem[q]).wait()
        pltpu.make_async_copy(col_hbm.at[pl.ds(cbase + g * IB, IB)], colb.at[q], isem[q]).wait()
        pltpu.make_async_copy(ew_hbm.at[pl.ds(eb, IBE)], ewb.at[q], isem[q]).wait()

    def adjust(q, toff):
        for jj in range(IBE // 16):
            rowb[q, pl.ds(jj * 16, 16)] = rowb[q, pl.ds(jj * 16, 16)] + toff

    def issue_gather(q, k2, p):
        pltpu.async_copy(tab_hbm.at[rowb.at[q, pl.ds(k2 * CHUNK, CHUNK)]],
                         gbuf.at[p], gsem[p])

    def wait_gather(q, k2, p):
        pltpu.make_async_copy(tab_hbm.at[rowb.at[q, pl.ds(k2 * CHUNK, CHUNK)]],
                              gbuf.at[p], gsem[p]).wait()

    def issue_scatter(q, k, p):
        pltpu.async_copy(gbuf.at[p], aggsh.at[colb.at[q, k]], ssem[p], add=True)

    def wait_scatter(q, k, p):
        pltpu.make_async_copy(gbuf.at[p], aggsh.at[colb.at[q, k]], ssem[p]).wait()

    def timestep(tl, tcarry):
        toff = c * (T_PER_CORE * N) + tl * N
        for kk in range(2):
            k = s + kk * NS
            @pl.when(k < NSLAB)
            def _():
                pltpu.sync_copy(tab_hbm.at[pl.ds(toff + k * RSLAB, RSLAB)], slab)
                pltpu.sync_copy(slab, aggsh.at[pl.ds(k * RSLAB, RSLAB)])
        plsc.subcore_barrier()

        issue_idx(0, 0)
        wait_idx(0, 0)
        adjust(0, toff)
        issue_gather(0, 0, 0)
        issue_gather(0, 1, 1)

        def pair(g2, carry):
            for gg in range(2):
                g = g2 * 2 + gg
                q = 1 - gg
                for k in range(8):
                    m = g * 8 + k
                    p = k % 4
                    pb = (k + 2) % 4
                    if k == 2:
                        if gg == 0:
                            issue_idx(g + 1, q)
                        else:
                            @pl.when(g2 < NGP2 - 1)
                            def _():
                                issue_idx(g + 1, q)
                    if k == 6:
                        if gg == 0:
                            wait_idx(g + 1, q)
                            adjust(q, toff)
                        else:
                            @pl.when(g2 < NGP2 - 1)
                            def _():
                                wait_idx(g + 1, q)
                                adjust(q, toff)
                    @pl.when(m >= 2)
                    def _():
                        wait_scatter(gg, (k + 2) % 8, pb)
                    @pl.when(m + 2 < CPT)
                    def _():
                        if k < 6:
                            issue_gather(gg, k + 2, pb)
                        else:
                            issue_gather(q, k - 6, pb)
                    wait_gather(gg, k, p)

                    def scale(e16, cc):
                        wv = ewb[gg, pl.ds(k * CHUNK + e16 * 16, 16)]
                        for l in range(16):
                            e = e16 * 16 + l
                            w = wv[l]
                            for jj in range(D // 16):
                                gbuf[p, e, pl.ds(jj * 16, 16)] = (
                                    gbuf[p, e, pl.ds(jj * 16, 16)] * w)
                        return cc

                    lax.fori_loop(0, CHUNK // 16, scale, 0)
                    issue_scatter(gg, k, p)
            return carry

        lax.fori_loop(0, NGP2, pair, 0)
        wait_scatter(1, 6, 2)
        wait_scatter(1, 7, 3)
        plsc.subcore_barrier()
        for kk in range(2):
            k = s + kk * NS
            @pl.when(k < NSLAB)
            def _():
                pltpu.sync_copy(aggsh.at[pl.ds(k * RSLAB, RSLAB)], slab)
                pltpu.sync_copy(slab, out_hbm.at[pl.ds(toff + k * RSLAB, RSLAB)])
        plsc.subcore_barrier()
        return tcarry

    lax.fori_loop(0, T_PER_CORE, timestep, 0)


BR = 2000
NB = T * N // BR
NBN = N // BR


def _prep_body(x_ref, w1_ref, dinv_ref, o_ref):
    h = jnp.dot(x_ref[...], w1_ref[...], preferred_element_type=jnp.float32)
    o_ref[...] = h * dinv_ref[...]


def _prep_call(xf, W1, dinv):
    return pl.pallas_call(
        _prep_body,
        grid=(NB,),
        in_specs=[
            pl.BlockSpec((BR, F_IN), lambda i: (i, 0)),
            pl.BlockSpec((F_IN, D), lambda i: (0, 0)),
            pl.BlockSpec((BR, 1), lambda i: (i % NBN, 0)),
        ],
        out_specs=pl.BlockSpec((BR, D), lambda i: (i, 0)),
        out_shape=jax.ShapeDtypeStruct((T * N, D), jnp.float32),
    )(xf, W1, dinv)


def _mid_body(a_ref, dinv_ref, b1_ref, w2_ref, o_ref):
    h = jnp.maximum(a_ref[...] * dinv_ref[...] + b1_ref[...], 0.0)
    o_ref[...] = jnp.dot(h, w2_ref[...], preferred_element_type=jnp.float32) * dinv_ref[...]


def _mid_call(a1, dinv, b1, W2):
    return pl.pallas_call(
        _mid_body,
        grid=(NB,),
        in_specs=[
            pl.BlockSpec((BR, D), lambda i: (i, 0)),
            pl.BlockSpec((BR, 1), lambda i: (i % NBN, 0)),
            pl.BlockSpec((1, D), lambda i: (0, 0)),
            pl.BlockSpec((D, D), lambda i: (0, 0)),
        ],
        out_specs=pl.BlockSpec((BR, D), lambda i: (i, 0)),
        out_shape=jax.ShapeDtypeStruct((T * N, D), jnp.float32),
    )(a1, dinv, b1, W2)


def _final_body(a_ref, dinv_ref, b2_ref, bt_ref, wi_ref, wh_ref, bl_ref,
                wm1_ref, bm1_ref, wm2_ref, bm2_ref, o_ref):
    dv = dinv_ref[...]
    add = b2_ref[...] + bt_ref[...]
    h = jnp.zeros((BR, H), jnp.float32)
    cst = jnp.zeros((BR, H), jnp.float32)
    wi = wi_ref[...]
    wh = wh_ref[...]
    bl = bl_ref[...]
    for t in range(T):
        et = a_ref[t] * dv + add
        g = (jnp.dot(et, wi, preferred_element_type=jnp.float32)
             + jnp.dot(h, wh, preferred_element_type=jnp.float32) + bl)
        gi = jax.nn.sigmoid(g[:, 0:H])
        gf = jax.nn.sigmoid(g[:, H:2 * H])
        gg = jnp.tanh(g[:, 2 * H:3 * H])
        go = jax.nn.sigmoid(g[:, 3 * H:4 * H])
        cst = gf * cst + gi * gg
        h = go * jnp.tanh(cst)
    z = jnp.maximum(jnp.dot(h, wm1_ref[...], preferred_element_type=jnp.float32)
                    + bm1_ref[...], 0.0)
    o_ref[...] = jnp.dot(z, wm2_ref[...], preferred_element_type=jnp.float32) + bm2_ref[...]


def _final_call(a2, dinv, b2, bias_table, Wi, Wh, b_lstm, Wm1, bm1, Wm2, bm2):
    return pl.pallas_call(
        _final_body,
        grid=(NBN,),
        in_specs=[
            pl.BlockSpec((T, BR, D), lambda i: (0, i, 0)),
            pl.BlockSpec((BR, 1), lambda i: (i, 0)),
            pl.BlockSpec((1, D), lambda i: (0, 0)),
            pl.BlockSpec((BR, D), lambda i: (i, 0)),
            pl.BlockSpec((D, 4 * H), lambda i: (0, 0)),
            pl.BlockSpec((H, 4 * H), lambda i: (0, 0)),
            pl.BlockSpec((1, 4 * H), lambda i: (0, 0)),
            pl.BlockSpec((H, D), lambda i: (0, 0)),
            pl.BlockSpec((1, D), lambda i: (0, 0)),
            pl.BlockSpec((D, 1), lambda i: (0, 0)),
            pl.BlockSpec((1, 1), lambda i: (0, 0)),
        ],
        out_specs=pl.BlockSpec((BR, 1), lambda i: (i, 0)),
        out_shape=jax.ShapeDtypeStruct((N, 1), jnp.float32),
    )(a2, dinv, b2, bias_table, Wi, Wh, b_lstm, Wm1, bm1, Wm2, bm2)


def kernel(x, edge_index, edge_weight, W1, b1, W2, b2, bias_table,
           Wi, Wh, b_lstm, Wm1, bm1, Wm2, bm2):
    row = jnp.concatenate([edge_index[0], jnp.zeros((EP - E,), jnp.int32)])
    col = jnp.concatenate([edge_index[1], jnp.zeros((EP - E,), jnp.int32)])
    ew = jnp.concatenate([edge_weight, jnp.zeros((EP - E,), jnp.float32)])
    col2d = col.reshape(NCHUNKS, CHUNK)
    deg_parts = _deg_kernel(col, ew).reshape(NC, NP1)
    deg = (deg_parts[0] + deg_parts[1] + 1.0)[:N]
    dinv = jnp.where(deg > 0, lax.rsqrt(jnp.maximum(deg, 1e-12)), 0.0)[:, None]

    xf = x.reshape(T * N, F_IN)
    hs1 = _prep_call(xf, W1, dinv)
    a1 = _agg_kernel(hs1, row, col2d, ew)
    hs2 = _mid_call(a1, dinv, b1.reshape(1, D), W2)
    a2 = _agg_kernel(hs2, row, col2d, ew)
    out = _final_call(a2.reshape(T, N, D), dinv, b2.reshape(1, D), bias_table,
                      Wi, Wh, b_lstm.reshape(1, 4 * H), Wm1, bm1.reshape(1, H // 2),
                      Wm2, bm2.reshape(1, 1))
    return out[:, 0]
